# baseline (speedup 1.0000x reference)
import numpy as np
import jax
import jax.numpy as jnp
from jax.experimental import pallas as pl

_R, _C = 128, 100000
_BR = 8

_U = np.uint32
_K1 = _U(42)
_K2 = _U(0 ^ 42 ^ 0x1BD11BDA)

_ROTS = (13, 15, 26, 6, 17, 29, 16, 24, 13, 15, 26, 6, 17, 29, 16, 24,
         13, 15, 26, 6)
_INJ = {
    4: (_K1, _U(_K2 + _U(1))),
    8: (_K2, _U(2)),
    12: (None, _U(_K1 + _U(3))),
    16: (_K1, _U(_K2 + _U(4))),
    20: (_K2, _U(5)),
}


def _rotl(x, d):
    return (x << _U(d)) | (x >> _U(32 - d))


def _body(x_ref, o_ref):
    pid = pl.program_id(0)
    logits = x_ref[...]
    shape = logits.shape
    row = jax.lax.broadcasted_iota(jnp.int32, shape, 0) + pid * _BR
    col = jax.lax.broadcasted_iota(jnp.int32, shape, 1)
    c1 = (row * _C + col).astype(jnp.uint32)

    x1 = c1 + _K1
    x0 = x1
    x1 = x0 ^ _rotl(x1, _ROTS[0])
    for rnd, r in enumerate(_ROTS[1:], start=2):
        x0 = x0 + x1
        x1 = x0 ^ _rotl(x1, r)
        if rnd in _INJ:
            a, b = _INJ[rnd]
            if a is not None:
                x0 = x0 + a
            x1 = x1 + b
    bits = x0 ^ x1

    fb = (bits >> _U(9)) | _U(0x3F800000)
    f = jax.lax.bitcast_convert_type(fb, jnp.float32)
    thr = jnp.exp(-jnp.exp(logits))
    o_ref[...] = ((2.0 - f) > thr).astype(jnp.float32)


@jax.jit
def kernel(logits):
    return pl.pallas_call(
        _body,
        out_shape=jax.ShapeDtypeStruct((_R, _C), jnp.float32),
        grid=(_R // _BR,),
        in_specs=[pl.BlockSpec((_BR, _C), lambda i: (i, 0))],
        out_specs=pl.BlockSpec((_BR, _C), lambda i: (i, 0)),
    )(logits)


# speedup vs baseline: 1.0007x; 1.0007x over previous
import numpy as np
import jax
import jax.numpy as jnp
from jax.experimental import pallas as pl
from jax.experimental.pallas import tpu as pltpu

_R, _C = 128, 100000
_BR = 8

_U = np.uint32
_K1 = _U(42)
_K2 = _U(0 ^ 42 ^ 0x1BD11BDA)

_ROTS = (13, 15, 26, 6, 17, 29, 16, 24, 13, 15, 26, 6, 17, 29, 16, 24,
         13, 15, 26, 6)
_INJ = {
    4: (_K1, _U(_K2 + _U(1))),
    8: (_K2, _U(2)),
    12: (None, _U(_K1 + _U(3))),
    16: (_K1, _U(_K2 + _U(4))),
    20: (_K2, _U(5)),
}


def _rotl(x, d):
    return (x << _U(d)) | (x >> _U(32 - d))


def _body(x_ref, o_ref):
    pid = pl.program_id(0)
    logits = x_ref[...]
    shape = logits.shape
    row = jax.lax.broadcasted_iota(jnp.int32, shape, 0) + pid * _BR
    col = jax.lax.broadcasted_iota(jnp.int32, shape, 1)
    c1 = (row * _C + col).astype(jnp.uint32)

    x1 = c1 + _K1
    x0 = x1
    x1 = x0 ^ _rotl(x1, _ROTS[0])
    for rnd, r in enumerate(_ROTS[1:], start=2):
        x0 = x0 + x1
        x1 = x0 ^ _rotl(x1, r)
        if rnd in _INJ:
            a, b = _INJ[rnd]
            if a is not None:
                x0 = x0 + a
            x1 = x1 + b
    bits = x0 ^ x1

    fb = (bits >> _U(9)) | _U(0x3F800000)
    f = jax.lax.bitcast_convert_type(fb, jnp.float32)
    thr = jnp.exp(-jnp.exp(logits))
    o_ref[...] = ((2.0 - f) > thr).astype(jnp.float32)


@jax.jit
def kernel(logits):
    return pl.pallas_call(
        _body,
        out_shape=jax.ShapeDtypeStruct((_R, _C), jnp.float32),
        grid=(_R // _BR,),
        in_specs=[pl.BlockSpec((_BR, _C), lambda i: (i, 0))],
        out_specs=pl.BlockSpec((_BR, _C), lambda i: (i, 0)),
        compiler_params=pltpu.CompilerParams(
            dimension_semantics=("parallel",)),
    )(logits)
